# baseline (speedup 1.0000x reference)
import functools

import jax
import jax.numpy as jnp
from jax import lax
from jax.experimental import pallas as pl
from jax.experimental.pallas import tpu as pltpu
from jax.experimental.pallas import tpu_sc as plsc

N_DST = 10000
D = 128
E_EDGES = 320000
LANES = 128
N_SUB = 16
ROWS = E_EDGES // LANES
ROWS_PER_TILE = (-(-ROWS // (N_SUB * 8))) * 8
ROWS_PAD = ROWS_PER_TILE * N_SUB
N_PAD = 10112
SLICE = N_PAD // N_SUB
DUMMY_DST = N_DST + 8
CH = 8
N_CHUNK = ROWS_PER_TILE // CH


def _sc_body(x_user, x_item, comb_a, comb_b, zf, zd,
             agg_a, deg_a, agg_b, deg_b,
             idx_v, rows_v, deg_v, agg_sp,
             gsem_a, gsem_b, ssem_a, ssem_b, isem_0, isem_1):
  c = lax.axis_index("c")
  s = lax.axis_index("s")
  gsem = (gsem_a, gsem_b)
  ssem = (ssem_a, ssem_b)
  isem = (isem_0, isem_1)
  ones16 = jnp.ones((16,), jnp.float32)

  def run(x_src, comb_h, agg_out, deg_out):
    pltpu.sync_copy(zd, deg_v)
    base = s * ROWS_PER_TILE

    def idx_load(buf, chunk):
      ch = jnp.minimum(chunk, N_CHUNK - 1)
      pltpu.async_copy(comb_h.at[pl.ds(base + ch * CH, CH)], idx_v.at[buf],
                       isem[buf])

    def idx_wait(buf):
      pltpu.make_async_copy(comb_h.at[pl.ds(base, CH)], idx_v.at[buf],
                            isem[buf]).wait()

    def g_fire(side, ibuf, r):
      pltpu.async_copy(x_src.at[idx_v.at[ibuf, r, 0]], rows_v.at[side],
                       gsem[side])

    def g_drain(side):
      pltpu.make_async_copy(x_src.at[idx_v.at[0, 0, 0]], rows_v.at[side],
                            gsem[side]).wait()

    def s_fire(side, ibuf, r):
      pltpu.async_copy(rows_v.at[side], agg_sp.at[idx_v.at[ibuf, r, 1]],
                       ssem[side], add=True)

    def s_drain(side):
      pltpu.make_async_copy(rows_v.at[side], agg_sp.at[idx_v.at[0, 0, 1]],
                            ssem[side]).wait()

    def count_deg(ibuf, r):
      for k in range(LANES // 16):
        idx16 = idx_v[ibuf, r, 1, pl.ds(k * 16, 16)]
        plsc.addupdate_scatter(deg_v, [idx16], ones16)

    pltpu.sync_copy(zf, agg_sp.at[pl.ds(s * SLICE, SLICE)])
    plsc.subcore_barrier()

    pltpu.sync_copy(comb_h.at[pl.ds(base, CH)], idx_v.at[0])
    g_fire(0, 0, 0)
    idx_load(1, 1)

    def chunk_rows(side, ibuf, next_first):
      for r in range(CH):
        other = 1 - side
        if r < CH - 1:
          g_fire(other, ibuf, r + 1)
        else:
          next_first(other)
        g_drain(side)
        s_fire(side, ibuf, r)
        count_deg(ibuf, r)
        s_drain(side)
        side = other
      return side

    def step(j, carry):
      side = 0

      def into_chunk1(other):
        idx_wait(1)
        g_fire(other, 1, 0)

      side = chunk_rows(side, 0, into_chunk1)
      idx_load(0, 2 * j + 2)

      def into_chunk0(other):
        idx_wait(0)
        g_fire(other, 0, 0)

      chunk_rows(side, 1, into_chunk0)
      idx_load(1, 2 * j + 3)
      return carry

    lax.fori_loop(0, N_CHUNK // 2, step, 0)
    g_drain(0)
    idx_wait(1)
    pltpu.sync_copy(deg_v, deg_out.at[s])
    plsc.subcore_barrier()
    pltpu.sync_copy(agg_sp.at[pl.ds(s * SLICE, SLICE)],
                    agg_out.at[pl.ds(s * SLICE, SLICE)])

  @pl.when(c == 0)
  def _():
    run(x_user, comb_a, agg_a, deg_a)

  @pl.when(c == 1)
  def _():
    run(x_item, comb_b, agg_b, deg_b)


_sc_call = functools.partial(
    pl.kernel,
    out_type=[
        jax.ShapeDtypeStruct((N_PAD, D), jnp.float32),
        jax.ShapeDtypeStruct((N_SUB, N_PAD), jnp.float32),
        jax.ShapeDtypeStruct((N_PAD, D), jnp.float32),
        jax.ShapeDtypeStruct((N_SUB, N_PAD), jnp.float32),
    ],
    mesh=plsc.VectorSubcoreMesh(core_axis_name="c", subcore_axis_name="s"),
    compiler_params=pltpu.CompilerParams(use_tc_tiling_on_sc=False,
                                         needs_layout_passes=False),
    scratch_types=[
        pltpu.VMEM((2, CH, 2, LANES), jnp.int32),
        pltpu.VMEM((2, LANES, D), jnp.float32),
        pltpu.VMEM((N_PAD,), jnp.float32),
        pltpu.VMEM_SHARED((N_PAD, D), jnp.float32),
        pltpu.SemaphoreType.DMA,
        pltpu.SemaphoreType.DMA,
        pltpu.SemaphoreType.DMA,
        pltpu.SemaphoreType.DMA,
        pltpu.SemaphoreType.DMA,
        pltpu.SemaphoreType.DMA,
    ],
)(_sc_body)


def _tc_self_body(x_i, ws_a, b_a, x_u, ws_b, b_b, self_i, self_u):
  self_i[...] = jnp.dot(x_i[...], ws_a[...],
                        preferred_element_type=jnp.float32) + b_a[...]
  self_u[...] = jnp.dot(x_u[...], ws_b[...],
                        preferred_element_type=jnp.float32) + b_b[...]


def _tc_body(self_i, agg_i, deg_i, self_u, agg_u, deg_u,
             wn_a, wn_b, out_i, out_u):
  def sage(base, agg, deg, wn):
    d = jnp.sum(deg[...], axis=1, keepdims=True)
    mean = agg[...] / jnp.maximum(d, 1.0)
    return base[...] + jnp.dot(mean, wn[...],
                               preferred_element_type=jnp.float32)

  out_i[...] = sage(self_i, agg_i, deg_i, wn_a)
  out_u[...] = sage(self_u, agg_u, deg_u, wn_b)


_TC_BLK = 1000


def _tc_self_call(x_i, ws_a, b_a, x_u, ws_b, b_b):
  row = lambda i: (i, 0)
  fix = lambda i: (0, 0)
  return pl.pallas_call(
      _tc_self_body,
      grid=(N_DST // _TC_BLK,),
      in_specs=[
          pl.BlockSpec((_TC_BLK, D), row),
          pl.BlockSpec((D, D), fix),
          pl.BlockSpec((1, D), fix),
          pl.BlockSpec((_TC_BLK, D), row),
          pl.BlockSpec((D, D), fix),
          pl.BlockSpec((1, D), fix),
      ],
      out_specs=[pl.BlockSpec((_TC_BLK, D), row),
                 pl.BlockSpec((_TC_BLK, D), row)],
      out_shape=[jax.ShapeDtypeStruct((N_DST, D), jnp.float32)] * 2,
  )(x_i, ws_a, b_a, x_u, ws_b, b_b)


def _tc_call(self_i, agg_i, deg_i, self_u, agg_u, deg_u, wn_a, wn_b):
  row = lambda i: (i, 0)
  fix = lambda i: (0, 0)
  return pl.pallas_call(
      _tc_body,
      grid=(N_DST // _TC_BLK,),
      in_specs=[
          pl.BlockSpec((_TC_BLK, D), row),
          pl.BlockSpec((_TC_BLK, D), row),
          pl.BlockSpec((_TC_BLK, N_SUB), row),
          pl.BlockSpec((_TC_BLK, D), row),
          pl.BlockSpec((_TC_BLK, D), row),
          pl.BlockSpec((_TC_BLK, N_SUB), row),
          pl.BlockSpec((D, D), fix),
          pl.BlockSpec((D, D), fix),
      ],
      out_specs=[pl.BlockSpec((_TC_BLK, D), row),
                 pl.BlockSpec((_TC_BLK, D), row)],
      out_shape=[jax.ShapeDtypeStruct((N_DST, D), jnp.float32)] * 2,
  )(self_i, agg_i, deg_i, self_u, agg_u, deg_u, wn_a, wn_b)


def _pad_edges(ei):
  n_pad = ROWS_PAD * LANES - E_EDGES
  src = jnp.concatenate(
      [ei[0].astype(jnp.int32), jnp.zeros((n_pad,), jnp.int32)])
  dst = jnp.concatenate(
      [ei[1].astype(jnp.int32), jnp.full((n_pad,), DUMMY_DST, jnp.int32)])
  return jnp.stack(
      [src.reshape(ROWS_PAD, LANES), dst.reshape(ROWS_PAD, LANES)], axis=1)


def kernel(x_user, x_item, edge_index_user_clicks_item,
           edge_index_item_rev_clicks_user, W_self_u2i, W_neigh_u2i, b_u2i,
           W_self_i2u, W_neigh_i2u, b_i2u):
  comb_a = _pad_edges(edge_index_user_clicks_item)
  comb_b = _pad_edges(edge_index_item_rev_clicks_user)
  zf = jnp.zeros((SLICE, D), jnp.float32)
  zd = jnp.zeros((N_PAD,), jnp.float32)
  self_i, self_u = _tc_self_call(
      x_item, W_self_u2i, b_u2i.reshape(1, D),
      x_user, W_self_i2u, b_i2u.reshape(1, D))
  agg_i, deg_i, agg_u, deg_u = _sc_call(
      x_user, x_item, comb_a, comb_b, zf, zd)
  out_item, out_user = _tc_call(
      self_i, agg_i, deg_i.T, self_u, agg_u, deg_u.T,
      W_neigh_u2i, W_neigh_i2u)
  return (out_item, out_user)


# speedup vs baseline: 1.0058x; 1.0058x over previous
import functools

import jax
import jax.numpy as jnp
from jax import lax
from jax.experimental import pallas as pl
from jax.experimental.pallas import tpu as pltpu
from jax.experimental.pallas import tpu_sc as plsc

N_DST = 10000
D = 128
E_EDGES = 320000
LANES = 128
N_SUB = 16
ROWS = E_EDGES // LANES
ROWS_PER_TILE = (-(-ROWS // (N_SUB * 8))) * 8
ROWS_PAD = ROWS_PER_TILE * N_SUB
N_PAD = 10112
SLICE = N_PAD // N_SUB
DUMMY_DST = N_DST + 8
CH = 8
N_CHUNK = ROWS_PER_TILE // CH


def _sc_body(x_user, x_item, comb_a, comb_b, zf, zd,
             agg_a, deg_a, agg_b, deg_b,
             idx_v, rows_v, deg_v, agg_sp,
             gsem_a, gsem_b, ssem_a, ssem_b, isem_0, isem_1):
  c = lax.axis_index("c")
  s = lax.axis_index("s")
  gsem = (gsem_a, gsem_b)
  ssem = (ssem_a, ssem_b)
  isem = (isem_0, isem_1)
  ones16 = jnp.ones((16,), jnp.float32)

  def run(x_src, comb_h, agg_out, deg_out):
    pltpu.sync_copy(zd, deg_v)
    base = s * ROWS_PER_TILE

    def idx_load(buf, chunk):
      ch = jnp.minimum(chunk, N_CHUNK - 1)
      pltpu.async_copy(comb_h.at[pl.ds(base + ch * CH, CH)], idx_v.at[buf],
                       isem[buf])

    def idx_wait(buf):
      pltpu.make_async_copy(comb_h.at[pl.ds(base, CH)], idx_v.at[buf],
                            isem[buf]).wait()

    def g_fire(side, ibuf, r):
      pltpu.async_copy(x_src.at[idx_v.at[ibuf, r, 0]], rows_v.at[side],
                       gsem[side])

    def g_drain(side):
      pltpu.make_async_copy(x_src.at[idx_v.at[0, 0, 0]], rows_v.at[side],
                            gsem[side]).wait()

    def s_fire(side, ibuf, r):
      pltpu.async_copy(rows_v.at[side], agg_sp.at[idx_v.at[ibuf, r, 1]],
                       ssem[side], add=True)

    def s_drain(side):
      pltpu.make_async_copy(rows_v.at[side], agg_sp.at[idx_v.at[0, 0, 1]],
                            ssem[side]).wait()

    def count_deg(ibuf, r):
      for k in range(LANES // 16):
        idx16 = idx_v[ibuf, r, 1, pl.ds(k * 16, 16)]
        plsc.addupdate_scatter(deg_v, [idx16], ones16)

    pltpu.sync_copy(zf, agg_sp.at[pl.ds(s * SLICE, SLICE)])
    plsc.subcore_barrier()

    pltpu.sync_copy(comb_h.at[pl.ds(base, CH)], idx_v.at[0])
    g_fire(0, 0, 0)
    idx_load(1, 1)

    def chunk_rows(side, ibuf, next_first):
      for r in range(CH):
        other = 1 - side
        if r < CH - 1:
          g_fire(other, ibuf, r + 1)
        else:
          next_first(other)
        g_drain(side)
        s_fire(side, ibuf, r)
        count_deg(ibuf, r)
        s_drain(side)
        side = other
      return side

    def step(j, carry):
      side = 0

      def into_chunk1(other):
        idx_wait(1)
        g_fire(other, 1, 0)

      side = chunk_rows(side, 0, into_chunk1)
      idx_load(0, 2 * j + 2)

      def into_chunk0(other):
        idx_wait(0)
        g_fire(other, 0, 0)

      chunk_rows(side, 1, into_chunk0)
      idx_load(1, 2 * j + 3)
      return carry

    lax.fori_loop(0, N_CHUNK // 2, step, 0)
    g_drain(0)
    idx_wait(1)
    pltpu.sync_copy(deg_v, deg_out.at[s])
    plsc.subcore_barrier()
    pltpu.sync_copy(agg_sp.at[pl.ds(s * SLICE, SLICE)],
                    agg_out.at[pl.ds(s * SLICE, SLICE)])

  @pl.when(c == 0)
  def _():
    run(x_user, comb_a, agg_a, deg_a)

  @pl.when(c == 1)
  def _():
    run(x_item, comb_b, agg_b, deg_b)


_sc_call = functools.partial(
    pl.kernel,
    out_type=[
        jax.ShapeDtypeStruct((N_PAD, D), jnp.float32),
        jax.ShapeDtypeStruct((N_SUB, N_PAD), jnp.float32),
        jax.ShapeDtypeStruct((N_PAD, D), jnp.float32),
        jax.ShapeDtypeStruct((N_SUB, N_PAD), jnp.float32),
    ],
    mesh=plsc.VectorSubcoreMesh(core_axis_name="c", subcore_axis_name="s"),
    compiler_params=pltpu.CompilerParams(use_tc_tiling_on_sc=False,
                                         needs_layout_passes=False),
    scratch_types=[
        pltpu.VMEM((2, CH, 2, LANES), jnp.int32),
        pltpu.VMEM((2, LANES, D), jnp.float32),
        pltpu.VMEM((N_PAD,), jnp.float32),
        pltpu.VMEM_SHARED((N_PAD, D), jnp.float32),
        pltpu.SemaphoreType.DMA,
        pltpu.SemaphoreType.DMA,
        pltpu.SemaphoreType.DMA,
        pltpu.SemaphoreType.DMA,
        pltpu.SemaphoreType.DMA,
        pltpu.SemaphoreType.DMA,
    ],
)(_sc_body)


def _tc_body(x_i, agg_i, deg_i, x_u, agg_u, deg_u,
             ws_a, wn_a, b_a, ws_b, wn_b, b_b, out_i, out_u):
  def sage(x, agg, deg, ws, wn, b):
    d = jnp.sum(deg[...], axis=1, keepdims=True)
    mean = agg[...] / jnp.maximum(d, 1.0)
    return (jnp.dot(x[...], ws[...], preferred_element_type=jnp.float32)
            + jnp.dot(mean, wn[...], preferred_element_type=jnp.float32)
            + b[...])

  out_i[...] = sage(x_i, agg_i, deg_i, ws_a, wn_a, b_a)
  out_u[...] = sage(x_u, agg_u, deg_u, ws_b, wn_b, b_b)


_TC_BLK = 1000


def _tc_call(x_i, agg_i, deg_i, x_u, agg_u, deg_u,
             ws_a, wn_a, b_a, ws_b, wn_b, b_b):
  row = lambda i: (i, 0)
  fix = lambda i: (0, 0)
  return pl.pallas_call(
      _tc_body,
      grid=(N_DST // _TC_BLK,),
      in_specs=[
          pl.BlockSpec((_TC_BLK, D), row),
          pl.BlockSpec((_TC_BLK, D), row),
          pl.BlockSpec((_TC_BLK, N_SUB), row),
          pl.BlockSpec((_TC_BLK, D), row),
          pl.BlockSpec((_TC_BLK, D), row),
          pl.BlockSpec((_TC_BLK, N_SUB), row),
          pl.BlockSpec((D, D), fix),
          pl.BlockSpec((D, D), fix),
          pl.BlockSpec((1, D), fix),
          pl.BlockSpec((D, D), fix),
          pl.BlockSpec((D, D), fix),
          pl.BlockSpec((1, D), fix),
      ],
      out_specs=[pl.BlockSpec((_TC_BLK, D), row),
                 pl.BlockSpec((_TC_BLK, D), row)],
      out_shape=[jax.ShapeDtypeStruct((N_DST, D), jnp.float32)] * 2,
  )(x_i, agg_i, deg_i, x_u, agg_u, deg_u, ws_a, wn_a, b_a, ws_b, wn_b, b_b)


def _pad_edges(ei):
  n_pad = ROWS_PAD * LANES - E_EDGES
  src = jnp.concatenate(
      [ei[0].astype(jnp.int32), jnp.zeros((n_pad,), jnp.int32)])
  dst = jnp.concatenate(
      [ei[1].astype(jnp.int32), jnp.full((n_pad,), DUMMY_DST, jnp.int32)])
  return jnp.stack(
      [src.reshape(ROWS_PAD, LANES), dst.reshape(ROWS_PAD, LANES)], axis=1)


def kernel(x_user, x_item, edge_index_user_clicks_item,
           edge_index_item_rev_clicks_user, W_self_u2i, W_neigh_u2i, b_u2i,
           W_self_i2u, W_neigh_i2u, b_i2u):
  comb_a = _pad_edges(edge_index_user_clicks_item)
  comb_b = _pad_edges(edge_index_item_rev_clicks_user)
  zf = jnp.zeros((SLICE, D), jnp.float32)
  zd = jnp.zeros((N_PAD,), jnp.float32)
  agg_i, deg_i, agg_u, deg_u = _sc_call(
      x_user, x_item, comb_a, comb_b, zf, zd)
  out_item, out_user = _tc_call(
      x_item, agg_i, deg_i.T,
      x_user, agg_u, deg_u.T,
      W_self_u2i, W_neigh_u2i, b_u2i.reshape(1, D),
      W_self_i2u, W_neigh_i2u, b_i2u.reshape(1, D))
  return (out_item, out_user)
